# baseline (speedup 1.0000x reference)
import functools

import jax
import jax.numpy as jnp
from jax import lax
from jax.experimental import pallas as pl
from jax.experimental.pallas import tpu as pltpu
from jax.experimental.pallas import tpu_sc as plsc

F = 26
V1 = 100001
D = 16
B = 16384
DD = 13
K1 = F * D + DD

NC, NS = 2, 16
NW = NC * NS
ROWS_PER_W = B // NW
CHUNK_ROWS = 128
N_CHUNKS = ROWS_PER_W // CHUNK_ROWS
G_PER_CHUNK = CHUNK_ROWS * F
NSTREAM = G_PER_CHUNK // 128


def _sc_gather(idx_flat, emb_flat, lin_flat):
    mesh = plsc.VectorSubcoreMesh(
        core_axis_name="c", subcore_axis_name="s", num_cores=NC, num_subcores=NS)

    @functools.partial(
        pl.kernel,
        out_type=(
            jax.ShapeDtypeStruct((B * F, D), jnp.float32),
            jax.ShapeDtypeStruct((B * F,), jnp.float32),
        ),
        mesh=mesh,
        scratch_types=[
            pltpu.VMEM((G_PER_CHUNK,), jnp.int32),
            pltpu.VMEM((G_PER_CHUNK, D), jnp.float32),
            pltpu.VMEM((G_PER_CHUNK,), jnp.float32),
            pltpu.SemaphoreType.DMA,
            pltpu.SemaphoreType.DMA,
        ],
        compiler_params=pltpu.CompilerParams(use_tc_tiling_on_sc=False),
    )
    def k(idx_hbm, emb_hbm, lin_hbm, emb_out, lin_out, idx_v, rows_v, lin_v,
          sem_e, sem_l):
        wid = lax.axis_index("s") * NC + lax.axis_index("c")

        def chunk_body(c, carry):
            base = (wid * N_CHUNKS + c) * G_PER_CHUNK
            pltpu.sync_copy(idx_hbm.at[pl.ds(base, G_PER_CHUNK)], idx_v)

            def fire(j, carry2):
                s = pl.ds(j * 128, 128)
                pltpu.make_async_copy(
                    emb_hbm.at[idx_v.at[s]], rows_v.at[s], sem_e).start()
                pltpu.make_async_copy(
                    lin_hbm.at[idx_v.at[s]], lin_v.at[s], sem_l).start()
                return carry2

            lax.fori_loop(0, NSTREAM, fire, 0)

            def drain(j, carry2):
                s = pl.ds(j * 128, 128)
                pltpu.make_async_copy(
                    emb_hbm.at[idx_v.at[s]], rows_v.at[s], sem_e).wait()
                pltpu.make_async_copy(
                    lin_hbm.at[idx_v.at[s]], lin_v.at[s], sem_l).wait()
                return carry2

            lax.fori_loop(0, NSTREAM, drain, 0)

            pltpu.sync_copy(rows_v, emb_out.at[pl.ds(base, G_PER_CHUNK)])
            pltpu.sync_copy(lin_v, lin_out.at[pl.ds(base, G_PER_CHUNK)])
            return carry

        lax.fori_loop(0, N_CHUNKS, chunk_body, 0)

    return k(idx_flat, emb_flat, lin_flat)


RB = 512


def _mlp_body(e_ref, lv_ref, d_ref, a1e_ref, a1d_ref, c1_ref, a2_ref, c2_ref,
              w3_ref, ldw_ref, cadd_ref, o_ref):
    e = e_ref[...]
    dd = d_ref[...]
    h1 = jnp.maximum(
        jnp.dot(e, a1e_ref[...]) + jnp.dot(dd, a1d_ref[...]) + c1_ref[...], 0.0)
    h2 = jnp.maximum(jnp.dot(h1, a2_ref[...]) + c2_ref[...], 0.0)
    deep = jnp.sum(h2 * w3_ref[...], axis=1)
    ki = lax.broadcasted_iota(jnp.int32, (F * D, D), 0) % D
    di = lax.broadcasted_iota(jnp.int32, (F * D, D), 1)
    fsum = (ki == di).astype(jnp.float32)
    s = jnp.dot(e, fsum)
    fm = 0.5 * (jnp.sum(s * s, axis=1) - jnp.sum(e * e, axis=1))
    lin_s = jnp.sum(lv_ref[...], axis=1)
    ld = jnp.sum(dd * ldw_ref[...], axis=1)
    o_ref[...] = lin_s + ld + fm + deep + cadd_ref[0]


def _tc_mlp(e, lv, dense, a1e, a1d, c1, a2, c2, w3, ldw, cadd):
    grid = (B // RB,)
    return pl.pallas_call(
        _mlp_body,
        grid=grid,
        in_specs=[
            pl.BlockSpec((RB, F * D), lambda i: (i, 0)),
            pl.BlockSpec((RB, F), lambda i: (i, 0)),
            pl.BlockSpec((RB, DD), lambda i: (i, 0)),
            pl.BlockSpec((F * D, 128), lambda i: (0, 0)),
            pl.BlockSpec((DD, 128), lambda i: (0, 0)),
            pl.BlockSpec((1, 128), lambda i: (0, 0)),
            pl.BlockSpec((128, 64), lambda i: (0, 0)),
            pl.BlockSpec((1, 64), lambda i: (0, 0)),
            pl.BlockSpec((1, 64), lambda i: (0, 0)),
            pl.BlockSpec((1, DD), lambda i: (0, 0)),
            pl.BlockSpec(memory_space=pltpu.SMEM),
        ],
        out_specs=pl.BlockSpec((RB,), lambda i: (i,)),
        out_shape=jax.ShapeDtypeStruct((B,), jnp.float32),
        compiler_params=pltpu.CompilerParams(
            dimension_semantics=("parallel",)),
    )(e, lv, dense, a1e, a1d, c1, a2, c2, w3, ldw, cadd)


def kernel(sparse_inputs, dense_inputs, emb_tables, lin_tables, ld_W, ld_b,
           bn0_g, bn0_b, W1, b1, bn1_g, bn1_b, W2, b2, bn2_g, bn2_b,
           Wout, bout, bias):
    offs = (jnp.arange(F, dtype=jnp.int32) * V1)[None, :]
    idx = sparse_inputs.astype(jnp.int32) + offs
    idx_flat = idx.reshape(B * F)
    emb_flat = emb_tables.reshape(F * V1, D)
    lin_flat = lin_tables.reshape(F * V1)

    emb_g, lin_g = _sc_gather(idx_flat, emb_flat, lin_flat)
    e = emb_g.reshape(B, F * D)
    lv = lin_g.reshape(B, F)

    s0 = 1.0 / jnp.sqrt(1.0 + 1e-5)
    g0 = bn0_g * s0
    w1f = W1 * g0[None, :]
    b1f = b1 + W1 @ (bn0_b * s0)
    s1 = bn1_g * s0
    w1ff = w1f * s1[:, None]
    c1 = (b1f * s1 + bn1_b)[None, :]
    s2 = bn2_g * s0
    a2 = (W2 * s2[:, None]).T
    c2 = (b2 * s2 + bn2_b)[None, :]
    a1e = w1ff[:, :F * D].T
    a1d = w1ff[:, F * D:].T
    w3 = Wout
    ldw = ld_W
    cadd = (bias + ld_b + bout).reshape(1)

    return _tc_mlp(e, lv, dense_inputs, a1e, a1d, c1, a2, c2, w3, ldw, cadd)


# speedup vs baseline: 2.7808x; 2.7808x over previous
import functools

import jax
import jax.numpy as jnp
from jax import lax
from jax.experimental import pallas as pl
from jax.experimental.pallas import tpu as pltpu
from jax.experimental.pallas import tpu_sc as plsc

F = 26
V1 = 100001
D = 16
B = 16384
DD = 13
FD = F * D

NC, NS = 2, 16
NW = NC * NS
COLS_PER_W = B // NW
N_CHUNKS = COLS_PER_W // 128


def _sc_gather(idx_t, emb_t, lin_t):
    mesh = plsc.VectorSubcoreMesh(
        core_axis_name="c", subcore_axis_name="s", num_cores=NC, num_subcores=NS)

    @functools.partial(
        pl.kernel,
        out_type=(
            jax.ShapeDtypeStruct((FD, B), jnp.float32),
            jax.ShapeDtypeStruct((B,), jnp.float32),
        ),
        mesh=mesh,
        scratch_types=[
            pltpu.VMEM((F, 128), jnp.int32),
            pltpu.VMEM((FD, 128), jnp.float32),
            pltpu.VMEM((F, 128), jnp.float32),
            pltpu.VMEM((128,), jnp.float32),
            pltpu.SemaphoreType.DMA,
            pltpu.SemaphoreType.DMA,
        ],
        compiler_params=pltpu.CompilerParams(use_tc_tiling_on_sc=False),
    )
    def k(idx_hbm, emb_hbm, lin_hbm, et_out, ls_out, idx_v, dst_v, lin_v,
          lsum_v, sem_e, sem_l):
        wid = lax.axis_index("s") * NC + lax.axis_index("c")

        def fire_lin(f, carry):
            pltpu.make_async_copy(
                lin_hbm.at[f, 0].at[idx_v.at[f]], lin_v.at[f], sem_l).start()
            return carry

        def drain_lin(f, carry):
            pltpu.make_async_copy(
                lin_hbm.at[f, 0].at[idx_v.at[f]], lin_v.at[f], sem_l).wait()
            return carry

        def fire_d(d, carry):
            def fire_f(f, carry2):
                pltpu.make_async_copy(
                    emb_hbm.at[f, d].at[idx_v.at[f]],
                    dst_v.at[f * D + d], sem_e).start()
                return carry2
            lax.fori_loop(0, F, fire_f, 0)
            return carry

        def drain_d(d, carry):
            def drain_f(f, carry2):
                pltpu.make_async_copy(
                    emb_hbm.at[f, d].at[idx_v.at[f]],
                    dst_v.at[f * D + d], sem_e).wait()
                return carry2
            lax.fori_loop(0, F, drain_f, 0)
            return carry

        def chunk_body(c, carry):
            col = wid * COLS_PER_W + c * 128
            pltpu.sync_copy(idx_hbm.at[:, pl.ds(col, 128)], idx_v)
            lax.fori_loop(0, F, fire_lin, 0)
            fire_d(0, 0)

            def roll(d, carry2):
                fire_d(d, 0)
                drain_d(d - 1, 0)
                return carry2

            lax.fori_loop(1, D, roll, 0)
            drain_d(D - 1, 0)
            lax.fori_loop(0, F, drain_lin, 0)

            for j in range(8):
                s = pl.ds(j * 16, 16)

                def acc_f(f, acc):
                    return acc + lin_v[f, s]

                lsum_v[s] = lax.fori_loop(
                    0, F, acc_f, jnp.zeros((16,), jnp.float32))

            pltpu.sync_copy(dst_v, et_out.at[:, pl.ds(col, 128)])
            pltpu.sync_copy(lsum_v, ls_out.at[pl.ds(col, 128)])
            return carry

        lax.fori_loop(0, N_CHUNKS, chunk_body, 0)

    return k(idx_t, emb_t, lin_t)


RB = 1024


def _mlp_body(e_ref, ls_ref, d_ref, a1e_ref, a1d_ref, c1_ref, a2_ref, c2_ref,
              w3_ref, ldw_ref, cadd_ref, o_ref):
    e = e_ref[...]
    dd = d_ref[...]
    h1 = jnp.maximum(
        jnp.dot(a1e_ref[...], e) + jnp.dot(a1d_ref[...], dd) + c1_ref[...],
        0.0)
    h2 = jnp.maximum(jnp.dot(a2_ref[...], h1) + c2_ref[...], 0.0)
    deep = jnp.sum(h2 * w3_ref[...], axis=0)
    ri = lax.broadcasted_iota(jnp.int32, (D, FD), 0)
    ki = lax.broadcasted_iota(jnp.int32, (D, FD), 1) % D
    fsum = (ri == ki).astype(jnp.float32)
    s = jnp.dot(fsum, e)
    fm = 0.5 * (jnp.sum(s * s, axis=0) - jnp.sum(e * e, axis=0))
    ld = jnp.sum(dd * ldw_ref[...], axis=0)
    o_ref[...] = ls_ref[...] + ld + fm + deep + cadd_ref[0]


def _tc_mlp(e_t, ls, dense_t, a1e, a1d, c1, a2, c2, w3, ldw, cadd):
    grid = (B // RB,)
    return pl.pallas_call(
        _mlp_body,
        grid=grid,
        in_specs=[
            pl.BlockSpec((FD, RB), lambda i: (0, i)),
            pl.BlockSpec((RB,), lambda i: (i,)),
            pl.BlockSpec((DD, RB), lambda i: (0, i)),
            pl.BlockSpec((128, FD), lambda i: (0, 0)),
            pl.BlockSpec((128, DD), lambda i: (0, 0)),
            pl.BlockSpec((128, 1), lambda i: (0, 0)),
            pl.BlockSpec((64, 128), lambda i: (0, 0)),
            pl.BlockSpec((64, 1), lambda i: (0, 0)),
            pl.BlockSpec((64, 1), lambda i: (0, 0)),
            pl.BlockSpec((DD, 1), lambda i: (0, 0)),
            pl.BlockSpec(memory_space=pltpu.SMEM),
        ],
        out_specs=pl.BlockSpec((RB,), lambda i: (i,)),
        out_shape=jax.ShapeDtypeStruct((B,), jnp.float32),
        compiler_params=pltpu.CompilerParams(
            dimension_semantics=("parallel",)),
    )(e_t, ls, dense_t, a1e, a1d, c1, a2, c2, w3, ldw, cadd)


def kernel(sparse_inputs, dense_inputs, emb_tables, lin_tables, ld_W, ld_b,
           bn0_g, bn0_b, W1, b1, bn1_g, bn1_b, W2, b2, bn2_g, bn2_b,
           Wout, bout, bias):
    idx_t = sparse_inputs.astype(jnp.int32).T
    emb_t = jnp.transpose(emb_tables, (0, 2, 1))
    lin_t = jnp.transpose(lin_tables, (0, 2, 1))
    dense_t = dense_inputs.T

    e_t, ls = _sc_gather(idx_t, emb_t, lin_t)

    s0 = 1.0 / jnp.sqrt(1.0 + 1e-5)
    g0 = bn0_g * s0
    w1f = W1 * g0[None, :]
    b1f = b1 + W1 @ (bn0_b * s0)
    s1 = bn1_g * s0
    w1ff = w1f * s1[:, None]
    c1 = (b1f * s1 + bn1_b)[:, None]
    s2 = bn2_g * s0
    a2 = W2 * s2[:, None]
    c2 = (b2 * s2 + bn2_b)[:, None]
    a1e = w1ff[:, :FD]
    a1d = w1ff[:, FD:]
    w3 = Wout.reshape(64, 1)
    ldw = ld_W.reshape(DD, 1)
    cadd = (bias + ld_b + bout).reshape(1)

    return _tc_mlp(e_t, ls, dense_t, a1e, a1d, c1, a2, c2, w3, ldw, cadd)


# speedup vs baseline: 2.8144x; 1.0121x over previous
import functools

import jax
import jax.numpy as jnp
from jax import lax
from jax.experimental import pallas as pl
from jax.experimental.pallas import tpu as pltpu
from jax.experimental.pallas import tpu_sc as plsc

F = 26
V1 = 100001
D = 16
B = 16384
DD = 13
FD = F * D

NC, NS = 2, 16
NW = NC * NS
CW = B // NW
NSLOT = 3


def _sc_gather(idx_t, emb_t, lin_t):
    mesh = plsc.VectorSubcoreMesh(
        core_axis_name="c", subcore_axis_name="s", num_cores=NC, num_subcores=NS)

    @functools.partial(
        pl.kernel,
        out_type=(
            jax.ShapeDtypeStruct((FD, B), jnp.float32),
            jax.ShapeDtypeStruct((B,), jnp.float32),
        ),
        mesh=mesh,
        scratch_types=[
            pltpu.VMEM((F, CW), jnp.int32),
            pltpu.VMEM((NSLOT, F, CW), jnp.float32),
            pltpu.VMEM((F, CW), jnp.float32),
            pltpu.VMEM((CW,), jnp.float32),
            pltpu.SemaphoreType.DMA,
            pltpu.SemaphoreType.DMA,
            pltpu.SemaphoreType.DMA,
        ],
        compiler_params=pltpu.CompilerParams(use_tc_tiling_on_sc=False),
    )
    def k(idx_hbm, emb_hbm, lin_hbm, et_out, ls_out, idx_v, gbuf, lbuf,
          lsum_v, sem_e, sem_l, sem_o):
        wid = lax.axis_index("s") * NC + lax.axis_index("c")
        col = wid * CW

        def fire_g(d, _):
            s = lax.rem(d, NSLOT)

            def f_body(f, carry):
                pltpu.make_async_copy(
                    emb_hbm.at[f, d].at[idx_v.at[f]], gbuf.at[s, f],
                    sem_e).start()
                return carry

            lax.fori_loop(0, F, f_body, 0)
            return _

        def drain_g(d, _):
            s = lax.rem(d, NSLOT)

            def f_body(f, carry):
                pltpu.make_async_copy(
                    emb_hbm.at[f, d].at[idx_v.at[f]], gbuf.at[s, f],
                    sem_e).wait()
                return carry

            lax.fori_loop(0, F, f_body, 0)
            return _

        def fire_out(d, _):
            s = lax.rem(d, NSLOT)

            def f_body(f, carry):
                pltpu.make_async_copy(
                    gbuf.at[s, f], et_out.at[f * D + d, pl.ds(col, CW)],
                    sem_o).start()
                return carry

            lax.fori_loop(0, F, f_body, 0)
            return _

        def drain_out(d, _):
            s = lax.rem(d, NSLOT)

            def f_body(f, carry):
                pltpu.make_async_copy(
                    gbuf.at[s, f], et_out.at[f * D + d, pl.ds(col, CW)],
                    sem_o).wait()
                return carry

            lax.fori_loop(0, F, f_body, 0)
            return _

        pltpu.sync_copy(idx_hbm.at[:, pl.ds(col, CW)], idx_v)

        def fire_lin(f, carry):
            pltpu.make_async_copy(
                lin_hbm.at[f, 0].at[idx_v.at[f]], lbuf.at[f], sem_l).start()
            return carry

        lax.fori_loop(0, F, fire_lin, 0)

        fire_g(0, 0)
        fire_g(1, 0)

        def dgroup(d, carry):
            drain_g(d, 0)

            def _do():
                drain_out(d - 1, 0)

            def _fg():
                fire_g(d + 2, 0)

            pl.when(d >= 1)(_do)
            fire_out(d, 0)
            pl.when(d + 2 < D)(_fg)
            return carry

        lax.fori_loop(0, D, dgroup, 0)
        drain_out(D - 1, 0)

        def drain_lin(f, carry):
            pltpu.make_async_copy(
                lin_hbm.at[f, 0].at[idx_v.at[f]], lbuf.at[f], sem_l).wait()
            return carry

        lax.fori_loop(0, F, drain_lin, 0)

        for j in range(CW // 16):
            sl = pl.ds(j * 16, 16)

            def acc_f(f, acc):
                return acc + lbuf[f, sl]

            lsum_v[sl] = lax.fori_loop(
                0, F, acc_f, jnp.zeros((16,), jnp.float32))

        pltpu.sync_copy(lsum_v, ls_out.at[pl.ds(col, CW)])

    return k(idx_t, emb_t, lin_t)


RB = 1024
NP = RB // 128


def _mlp_body(e_ref, ls_ref, d_ref, a1e_ref, a1d_ref, c1_ref, a2_ref, c2_ref,
              w3_ref, ldw_ref, cadd_ref, o_ref):
    ri = lax.broadcasted_iota(jnp.int32, (D, FD), 0)
    ki = lax.broadcasted_iota(jnp.int32, (D, FD), 1) % D
    fsum = (ri == ki).astype(jnp.float32)
    for p in range(NP):
        sl = pl.ds(p * 128, 128)
        e = e_ref[:, p, :]
        dd = d_ref[:, sl]
        h1 = jnp.maximum(
            jnp.dot(a1e_ref[...], e) + jnp.dot(a1d_ref[...], dd)
            + c1_ref[...], 0.0)
        h2 = jnp.maximum(jnp.dot(a2_ref[...], h1) + c2_ref[...], 0.0)
        deep = jnp.sum(h2 * w3_ref[...], axis=0)
        s = jnp.dot(fsum, e)
        fm = 0.5 * (jnp.sum(s * s, axis=0) - jnp.sum(e * e, axis=0))
        ld = jnp.sum(dd * ldw_ref[...], axis=0)
        o_ref[sl] = ls_ref[sl] + ld + fm + deep + cadd_ref[0]


def _tc_mlp(e3, ls, dense_t, a1e, a1d, c1, a2, c2, w3, ldw, cadd):
    grid = (B // RB,)
    return pl.pallas_call(
        _mlp_body,
        grid=grid,
        in_specs=[
            pl.BlockSpec((FD, NP, 128), lambda i: (0, i, 0)),
            pl.BlockSpec((RB,), lambda i: (i,)),
            pl.BlockSpec((DD, RB), lambda i: (0, i)),
            pl.BlockSpec((128, FD), lambda i: (0, 0)),
            pl.BlockSpec((128, DD), lambda i: (0, 0)),
            pl.BlockSpec((128, 1), lambda i: (0, 0)),
            pl.BlockSpec((64, 128), lambda i: (0, 0)),
            pl.BlockSpec((64, 1), lambda i: (0, 0)),
            pl.BlockSpec((64, 1), lambda i: (0, 0)),
            pl.BlockSpec((DD, 1), lambda i: (0, 0)),
            pl.BlockSpec(memory_space=pltpu.SMEM),
        ],
        out_specs=pl.BlockSpec((RB,), lambda i: (i,)),
        out_shape=jax.ShapeDtypeStruct((B,), jnp.float32),
        compiler_params=pltpu.CompilerParams(
            dimension_semantics=("parallel",)),
    )(e3, ls, dense_t, a1e, a1d, c1, a2, c2, w3, ldw, cadd)


def kernel(sparse_inputs, dense_inputs, emb_tables, lin_tables, ld_W, ld_b,
           bn0_g, bn0_b, W1, b1, bn1_g, bn1_b, W2, b2, bn2_g, bn2_b,
           Wout, bout, bias):
    idx_t = sparse_inputs.astype(jnp.int32).T
    emb_t = jnp.transpose(emb_tables, (0, 2, 1))
    lin_t = jnp.transpose(lin_tables, (0, 2, 1))
    dense_t = dense_inputs.T

    e_t, ls = _sc_gather(idx_t, emb_t, lin_t)
    e3 = e_t.reshape(FD, B // 128, 128)

    s0 = 1.0 / jnp.sqrt(1.0 + 1e-5)
    g0 = bn0_g * s0
    w1f = W1 * g0[None, :]
    b1f = b1 + W1 @ (bn0_b * s0)
    s1 = bn1_g * s0
    w1ff = w1f * s1[:, None]
    c1 = (b1f * s1 + bn1_b)[:, None]
    s2 = bn2_g * s0
    a2 = W2 * s2[:, None]
    c2 = (b2 * s2 + bn2_b)[:, None]
    a1e = w1ff[:, :FD]
    a1d = w1ff[:, FD:]
    w3 = Wout.reshape(64, 1)
    ldw = ld_W.reshape(DD, 1)
    cadd = (bias + ld_b + bout).reshape(1)

    return _tc_mlp(e3, ls, dense_t, a1e, a1d, c1, a2, c2, w3, ldw, cadd)
